# Initial kernel scaffold; baseline (speedup 1.0000x reference)
import jax
import jax.numpy as jnp
from jax.experimental import pallas as pl


def kernel(x, emb):
    raise NotImplementedError("write your pallas kernel here")


# speedup vs baseline: 1.1236x; 1.1236x over previous
import functools

import jax
import jax.numpy as jnp
from jax import lax
from jax.experimental import pallas as pl
from jax.experimental.pallas import tpu as pltpu
from jax.experimental.pallas import tpu_sc as plsc

_K = 8192
_C = 32
_N = 16384
_BN = 256
_NBLK = _N // _BN

_NW = 32
_BPW = _N // _NW
_CHUNK = 128
_J = _BPW // _CHUNK


def _tc_body(x_ref, emb_ref, x2_ref, e2_ref, idx_ref, loss_ref):
    xb = x_ref[...].astype(jnp.bfloat16)
    eb = emb_ref[...]
    xy = lax.dot_general(eb, xb, (((1,), (1,)), ((), ())),
                         preferred_element_type=jnp.float32)
    d2 = (x2_ref[...] + e2_ref[...]) - 2.0 * xy
    v = jnp.maximum(d2, 0.0)
    dist = jnp.where(v == 0.0, v, v * lax.rsqrt(v))
    m = jnp.min(dist, axis=0, keepdims=True)
    ii = lax.broadcasted_iota(jnp.int32, (_K, _BN), 0)
    idx = jnp.min(jnp.where(dist == m, ii, _K), axis=0)
    idx_ref[...] = idx.reshape(1, 1, _BN)

    @pl.when(pl.program_id(0) == 0)
    def _():
        loss_ref[...] = jnp.zeros_like(loss_ref)

    loss_ref[...] += jnp.sum(m * m).reshape(1, 1)


_tc_call = pl.pallas_call(
    _tc_body,
    grid=(_NBLK,),
    in_specs=[
        pl.BlockSpec((_BN, _C), lambda i: (i, 0)),
        pl.BlockSpec((_K, _C), lambda i: (0, 0)),
        pl.BlockSpec((1, _BN), lambda i: (0, i)),
        pl.BlockSpec((_K, 1), lambda i: (0, 0)),
    ],
    out_specs=[
        pl.BlockSpec((1, 1, _BN), lambda i: (i, 0, 0)),
        pl.BlockSpec((1, 1), lambda i: (0, 0)),
    ],
    out_shape=[
        jax.ShapeDtypeStruct((_NBLK, 1, _BN), jnp.int32),
        jax.ShapeDtypeStruct((1, 1), jnp.float32),
    ],
)


def _sc_gather_body(emb_hbm, idx_hbm, out_hbm, idx_v, rows_v, sem):
    wid = lax.axis_index("s") * 2 + lax.axis_index("c")
    pltpu.sync_copy(idx_hbm.at[wid], idx_v)
    copies = [
        pltpu.async_copy(emb_hbm.at[idx_v.at[j]], rows_v.at[j], sem)
        for j in range(_J)
    ]
    for cp in copies:
        cp.wait()
    for j in range(_J):
        pltpu.sync_copy(rows_v.at[j],
                        out_hbm.at[pl.ds(wid * _BPW + j * _CHUNK, _CHUNK)])


def _sc_gather(emb, idx):
    fn = pl.kernel(
        _sc_gather_body,
        mesh=plsc.VectorSubcoreMesh(core_axis_name="c", subcore_axis_name="s"),
        out_type=jax.ShapeDtypeStruct((_N, _C), jnp.float32),
        scratch_types=[
            pltpu.VMEM((_J, _CHUNK), jnp.int32),
            pltpu.VMEM((_J, _CHUNK, _C), jnp.float32),
            pltpu.SemaphoreType.DMA,
        ],
        compiler_params=pltpu.CompilerParams(use_tc_tiling_on_sc=False),
    )
    return fn(emb, idx)


def kernel(x, emb):
    B, C, H, W = x.shape
    xf = jnp.transpose(x, (0, 2, 3, 1)).reshape(-1, C)
    x2 = jnp.sum(xf ** 2, axis=1)[None, :]
    e2 = jnp.sum(emb ** 2, axis=1)[:, None]
    idx3, loss_sum = _tc_call(xf, emb, x2, e2)
    idx = idx3.reshape(_NW, _J, _CHUNK)
    quant = _sc_gather(emb, idx)
    loss = (2.0 / (_N * _C)) * loss_sum[0, 0]
    quant = xf + (quant - xf)
    quant_out = jnp.transpose(quant.reshape(B, H, W, C), (0, 3, 1, 2))
    return quant_out, loss, idx.reshape(B, H, W)


# speedup vs baseline: 1.1397x; 1.0144x over previous
import functools

import jax
import jax.numpy as jnp
from jax import lax
from jax.experimental import pallas as pl
from jax.experimental.pallas import tpu as pltpu
from jax.experimental.pallas import tpu_sc as plsc

_K = 8192
_C = 32
_N = 16384
_BN = 256
_NBLK = _N // _BN

_NW = 32
_BPW = _N // _NW
_CHUNK = 128
_J = _BPW // _CHUNK


def _tc_body(x_ref, emb_ref, x2_ref, e2_ref, idx_ref, loss_ref):
    xb = x_ref[...].astype(jnp.bfloat16)
    eb = emb_ref[...]
    xy = lax.dot_general(eb, xb, (((1,), (1,)), ((), ())),
                         preferred_element_type=jnp.float32)
    d2 = (x2_ref[...] + e2_ref[...]) - 2.0 * xy
    v = jnp.maximum(d2, 0.0)
    dist = jnp.where(v == 0.0, v, v * lax.rsqrt(v))
    q_sz = _K // 2
    acc_v = acc_vb = acc_i = None
    for q in range(2):
        dq = dist[q * q_sz:(q + 1) * q_sz, :]
        m_q = jnp.min(dq, axis=0)
        ii = lax.broadcasted_iota(jnp.int32, (q_sz, _BN), 0) + q * q_sz
        i_q = jnp.min(jnp.where(dq == m_q[None, :], ii, _K), axis=0)
        if q == 0:
            acc_v, acc_i = m_q, i_q
            acc_vb = m_q.astype(jnp.bfloat16).astype(jnp.float32)
        else:
            keep = acc_vb <= m_q
            acc_v = jnp.where(keep, acc_v, m_q)
            acc_i = jnp.where(keep, acc_i, i_q)
            acc_vb = jnp.where(keep, acc_vb,
                               m_q.astype(jnp.bfloat16).astype(jnp.float32))
    m = acc_v.reshape(1, _BN)
    idx_ref[...] = acc_i.reshape(1, 1, _BN)

    @pl.when(pl.program_id(0) == 0)
    def _():
        loss_ref[...] = jnp.zeros_like(loss_ref)

    loss_ref[...] += jnp.sum(m * m).reshape(1, 1)


_tc_call = pl.pallas_call(
    _tc_body,
    grid=(_NBLK,),
    in_specs=[
        pl.BlockSpec((_BN, _C), lambda i: (i, 0)),
        pl.BlockSpec((_K, _C), lambda i: (0, 0)),
        pl.BlockSpec((1, _BN), lambda i: (0, i)),
        pl.BlockSpec((_K, 1), lambda i: (0, 0)),
    ],
    out_specs=[
        pl.BlockSpec((1, 1, _BN), lambda i: (i, 0, 0)),
        pl.BlockSpec((1, 1), lambda i: (0, 0)),
    ],
    out_shape=[
        jax.ShapeDtypeStruct((_NBLK, 1, _BN), jnp.int32),
        jax.ShapeDtypeStruct((1, 1), jnp.float32),
    ],
)


def _sc_gather_body(emb_hbm, idx_hbm, out_hbm, idx_v, rows_v, sem):
    wid = lax.axis_index("s") * 2 + lax.axis_index("c")
    pltpu.sync_copy(idx_hbm.at[wid], idx_v)
    copies = [
        pltpu.async_copy(emb_hbm.at[idx_v.at[j]], rows_v.at[j], sem)
        for j in range(_J)
    ]
    for cp in copies:
        cp.wait()
    for j in range(_J):
        pltpu.sync_copy(rows_v.at[j],
                        out_hbm.at[pl.ds(wid * _BPW + j * _CHUNK, _CHUNK)])


def _sc_gather(emb, idx):
    fn = pl.kernel(
        _sc_gather_body,
        mesh=plsc.VectorSubcoreMesh(core_axis_name="c", subcore_axis_name="s"),
        out_type=jax.ShapeDtypeStruct((_N, _C), jnp.float32),
        scratch_types=[
            pltpu.VMEM((_J, _CHUNK), jnp.int32),
            pltpu.VMEM((_J, _CHUNK, _C), jnp.float32),
            pltpu.SemaphoreType.DMA,
        ],
        compiler_params=pltpu.CompilerParams(use_tc_tiling_on_sc=False),
    )
    return fn(emb, idx)


def kernel(x, emb):
    B, C, H, W = x.shape
    xf = jnp.transpose(x, (0, 2, 3, 1)).reshape(-1, C)
    x2 = jnp.sum(xf ** 2, axis=1)[None, :]
    e2 = jnp.sum(emb ** 2, axis=1)[:, None]
    idx3, loss_sum = _tc_call(xf, emb, x2, e2)
    idx = idx3.reshape(_NW, _J, _CHUNK)
    quant = _sc_gather(emb, idx)
    loss = (2.0 / (_N * _C)) * loss_sum[0, 0]
    quant = xf + (quant - xf)
    quant_out = jnp.transpose(quant.reshape(B, H, W, C), (0, 3, 1, 2))
    return quant_out, loss, idx.reshape(B, H, W)


# speedup vs baseline: 1.2491x; 1.0959x over previous
import functools

import jax
import jax.numpy as jnp
from jax import lax
from jax.experimental import pallas as pl
from jax.experimental.pallas import tpu as pltpu
from jax.experimental.pallas import tpu_sc as plsc

_K = 8192
_C = 32
_N = 16384
_BN = 512
_NBLK = _N // _BN

_NW = 32
_BPW = _N // _NW
_CHUNK = 128
_J = _BPW // _CHUNK


def _tc_body(x_ref, emb_ref, x2_ref, e2_ref, idx_ref, loss_ref):
    xb = x_ref[...].astype(jnp.bfloat16)
    eb = emb_ref[...]
    xy = lax.dot_general(eb, xb, (((1,), (1,)), ((), ())),
                         preferred_element_type=jnp.float32)
    d2 = (x2_ref[...] + e2_ref[...]) - 2.0 * xy
    v = jnp.maximum(d2, 0.0)
    dist = jnp.where(v == 0.0, v, v * lax.rsqrt(v))
    q_sz = _K // 2
    acc_v = acc_vb = acc_i = None
    for q in range(2):
        dq = dist[q * q_sz:(q + 1) * q_sz, :]
        m_q = jnp.min(dq, axis=0)
        ii = lax.broadcasted_iota(jnp.int32, (q_sz, _BN), 0) + q * q_sz
        i_q = jnp.min(jnp.where(dq == m_q[None, :], ii, _K), axis=0)
        if q == 0:
            acc_v, acc_i = m_q, i_q
            acc_vb = m_q.astype(jnp.bfloat16).astype(jnp.float32)
        else:
            keep = acc_vb <= m_q
            acc_v = jnp.where(keep, acc_v, m_q)
            acc_i = jnp.where(keep, acc_i, i_q)
            acc_vb = jnp.where(keep, acc_vb,
                               m_q.astype(jnp.bfloat16).astype(jnp.float32))
    m = acc_v.reshape(1, _BN)
    idx_ref[...] = acc_i.reshape(1, 1, _BN)

    @pl.when(pl.program_id(0) == 0)
    def _():
        loss_ref[...] = jnp.zeros_like(loss_ref)

    loss_ref[...] += jnp.sum(m * m).reshape(1, 1)


_tc_call = pl.pallas_call(
    _tc_body,
    grid=(_NBLK,),
    in_specs=[
        pl.BlockSpec((_BN, _C), lambda i: (i, 0)),
        pl.BlockSpec((_K, _C), lambda i: (0, 0)),
        pl.BlockSpec((1, _BN), lambda i: (0, i)),
        pl.BlockSpec((_K, 1), lambda i: (0, 0)),
    ],
    out_specs=[
        pl.BlockSpec((1, 1, _BN), lambda i: (i, 0, 0)),
        pl.BlockSpec((1, 1), lambda i: (0, 0)),
    ],
    out_shape=[
        jax.ShapeDtypeStruct((_NBLK, 1, _BN), jnp.int32),
        jax.ShapeDtypeStruct((1, 1), jnp.float32),
    ],
)


def _sc_gather_body(emb_hbm, idx_hbm, out_hbm, idx_v, rows_v, sem):
    wid = lax.axis_index("s") * 2 + lax.axis_index("c")
    pltpu.sync_copy(idx_hbm.at[wid], idx_v)
    copies = [
        pltpu.async_copy(emb_hbm.at[idx_v.at[j]], rows_v.at[j], sem)
        for j in range(_J)
    ]
    for cp in copies:
        cp.wait()
    for j in range(_J):
        pltpu.sync_copy(rows_v.at[j],
                        out_hbm.at[pl.ds(wid * _BPW + j * _CHUNK, _CHUNK)])


def _sc_gather(emb, idx):
    fn = pl.kernel(
        _sc_gather_body,
        mesh=plsc.VectorSubcoreMesh(core_axis_name="c", subcore_axis_name="s"),
        out_type=jax.ShapeDtypeStruct((_N, _C), jnp.float32),
        scratch_types=[
            pltpu.VMEM((_J, _CHUNK), jnp.int32),
            pltpu.VMEM((_J, _CHUNK, _C), jnp.float32),
            pltpu.SemaphoreType.DMA,
        ],
        compiler_params=pltpu.CompilerParams(use_tc_tiling_on_sc=False),
    )
    return fn(emb, idx)


def kernel(x, emb):
    B, C, H, W = x.shape
    xf = jnp.transpose(x, (0, 2, 3, 1)).reshape(-1, C)
    x2 = jnp.sum(xf ** 2, axis=1)[None, :]
    e2 = jnp.sum(emb ** 2, axis=1)[:, None]
    idx3, loss_sum = _tc_call(xf, emb, x2, e2)
    idx = idx3.reshape(_NW, _J, _CHUNK)
    quant = _sc_gather(emb, idx)
    loss = (2.0 / (_N * _C)) * loss_sum[0, 0]
    quant = xf + (quant - xf)
    quant_out = jnp.transpose(quant.reshape(B, H, W, C), (0, 3, 1, 2))
    return quant_out, loss, idx.reshape(B, H, W)


# speedup vs baseline: 1.2748x; 1.0206x over previous
import functools

import jax
import jax.numpy as jnp
from jax import lax
from jax.experimental import pallas as pl
from jax.experimental.pallas import tpu as pltpu
from jax.experimental.pallas import tpu_sc as plsc

_K = 8192
_C = 32
_N = 16384
_BN = 1024
_NBLK = _N // _BN

_NW = 32
_BPW = _N // _NW
_CHUNK = 128
_J = _BPW // _CHUNK


def _tc_body(x_ref, emb_ref, x2_ref, e2_ref, idx_ref, loss_ref):
    xb = x_ref[...].astype(jnp.bfloat16)
    eb = emb_ref[...]
    xy = lax.dot_general(eb, xb, (((1,), (1,)), ((), ())),
                         preferred_element_type=jnp.float32)
    d2 = (x2_ref[...] + e2_ref[...]) - 2.0 * xy
    v = jnp.maximum(d2, 0.0)
    dist = jnp.where(v == 0.0, v, v * lax.rsqrt(v))
    q_sz = _K // 2
    acc_v = acc_vb = acc_i = None
    for q in range(2):
        dq = dist[q * q_sz:(q + 1) * q_sz, :]
        m_q = jnp.min(dq, axis=0)
        ii = lax.broadcasted_iota(jnp.int32, (q_sz, _BN), 0) + q * q_sz
        i_q = jnp.min(jnp.where(dq == m_q[None, :], ii, _K), axis=0)
        if q == 0:
            acc_v, acc_i = m_q, i_q
            acc_vb = m_q.astype(jnp.bfloat16).astype(jnp.float32)
        else:
            keep = acc_vb <= m_q
            acc_v = jnp.where(keep, acc_v, m_q)
            acc_i = jnp.where(keep, acc_i, i_q)
            acc_vb = jnp.where(keep, acc_vb,
                               m_q.astype(jnp.bfloat16).astype(jnp.float32))
    m = acc_v.reshape(1, _BN)
    idx_ref[...] = acc_i.reshape(1, 1, _BN)

    @pl.when(pl.program_id(0) == 0)
    def _():
        loss_ref[...] = jnp.zeros_like(loss_ref)

    loss_ref[...] += jnp.sum(m * m).reshape(1, 1)


_tc_call = pl.pallas_call(
    _tc_body,
    grid=(_NBLK,),
    in_specs=[
        pl.BlockSpec((_BN, _C), lambda i: (i, 0)),
        pl.BlockSpec((_K, _C), lambda i: (0, 0)),
        pl.BlockSpec((1, _BN), lambda i: (0, i)),
        pl.BlockSpec((_K, 1), lambda i: (0, 0)),
    ],
    out_specs=[
        pl.BlockSpec((1, 1, _BN), lambda i: (i, 0, 0)),
        pl.BlockSpec((1, 1), lambda i: (0, 0)),
    ],
    out_shape=[
        jax.ShapeDtypeStruct((_NBLK, 1, _BN), jnp.int32),
        jax.ShapeDtypeStruct((1, 1), jnp.float32),
    ],
)


def _sc_gather_body(emb_hbm, idx_hbm, out_hbm, idx_v, rows_v, sem):
    wid = lax.axis_index("s") * 2 + lax.axis_index("c")
    pltpu.sync_copy(idx_hbm.at[wid], idx_v)
    copies = [
        pltpu.async_copy(emb_hbm.at[idx_v.at[j]], rows_v.at[j], sem)
        for j in range(_J)
    ]
    for cp in copies:
        cp.wait()
    for j in range(_J):
        pltpu.sync_copy(rows_v.at[j],
                        out_hbm.at[pl.ds(wid * _BPW + j * _CHUNK, _CHUNK)])


def _sc_gather(emb, idx):
    fn = pl.kernel(
        _sc_gather_body,
        mesh=plsc.VectorSubcoreMesh(core_axis_name="c", subcore_axis_name="s"),
        out_type=jax.ShapeDtypeStruct((_N, _C), jnp.float32),
        scratch_types=[
            pltpu.VMEM((_J, _CHUNK), jnp.int32),
            pltpu.VMEM((_J, _CHUNK, _C), jnp.float32),
            pltpu.SemaphoreType.DMA,
        ],
        compiler_params=pltpu.CompilerParams(use_tc_tiling_on_sc=False),
    )
    return fn(emb, idx)


def kernel(x, emb):
    B, C, H, W = x.shape
    xf = jnp.transpose(x, (0, 2, 3, 1)).reshape(-1, C)
    x2 = jnp.sum(xf ** 2, axis=1)[None, :]
    e2 = jnp.sum(emb ** 2, axis=1)[:, None]
    idx3, loss_sum = _tc_call(xf, emb, x2, e2)
    idx = idx3.reshape(_NW, _J, _CHUNK)
    quant = _sc_gather(emb, idx)
    loss = (2.0 / (_N * _C)) * loss_sum[0, 0]
    quant = xf + (quant - xf)
    quant_out = jnp.transpose(quant.reshape(B, H, W, C), (0, 3, 1, 2))
    return quant_out, loss, idx.reshape(B, H, W)
